# baseline (speedup 1.0000x reference)
import math
import functools

import jax
import jax.numpy as jnp
from jax import lax
from jax.experimental import pallas as pl
from jax.experimental.pallas import tpu as pltpu
from jax.experimental.pallas import tpu_sc as plsc

N_POINTS = 10000
IN_CH = 3
OUT_CH = 40
DIM_MODEL = [32, 64, 128, 256, 512]
K = 16
RATIO = 0.25


def _lin(p, x):
    return x @ p['W'] + p['b']


def _bn(p, x, eps=1e-5):
    mu = x.mean(0)
    var = x.var(0)
    return (x - mu) / jnp.sqrt(var + eps) * p['gamma'] + p['beta']


def _lin_bn_relu(p, x):
    return jax.nn.relu(_bn(p, x @ p['W'] + p['b']))


def _mlp2_relu(p, x):
    h = jax.nn.relu(x @ p['W1'] + p['b1'])
    return jax.nn.relu(h @ p['W2'] + p['b2'])


import numpy as np

_INF = np.float32(np.inf)
_BIG_I32 = np.int32(2**31 - 1)


def _knn_body(qx_ref, qy_ref, qz_ref, cx_ref, cy_ref, cz_ref, out_ref,
              *, k, qblk, exclude_self):
    n = cx_ref.shape[1]
    qx = qx_ref[...]
    qy = qy_ref[...]
    qz = qz_ref[...]
    cx = cx_ref[...]
    cy = cy_ref[...]
    cz = cz_ref[...]
    d = (qx - cx) ** 2 + (qy - cy) ** 2 + (qz - cz) ** 2
    ci = jax.lax.broadcasted_iota(jnp.int32, (qblk, n), 1)
    if exclude_self:
        s = pl.program_id(0) * qblk
        qi = jax.lax.broadcasted_iota(jnp.int32, (qblk, n), 0)
        d = jnp.where(ci == qi + s, _INF, d)
    for r in range(k):
        m = jnp.min(d, axis=1, keepdims=True)
        idx = jnp.min(jnp.where(d == m, ci, _BIG_I32), axis=1, keepdims=True)
        out_ref[:, r:r + 1] = idx
        d = jnp.where(ci == idx, _INF, d)


def _round_up(v, m):
    return ((v + m - 1) // m) * m


def _knn_pallas(pos_q, pos_c, k, exclude_self):
    nq = pos_q.shape[0]
    n = pos_c.shape[0]
    qblk = 1024 if nq >= 1024 else _round_up(nq, 8)
    nq_pad = _round_up(nq, qblk)
    q = jnp.pad(pos_q, ((0, nq_pad - nq), (0, 0)))
    qx = q[:, 0:1]
    qy = q[:, 1:2]
    qz = q[:, 2:3]
    cx = pos_c[:, 0].reshape(1, n)
    cy = pos_c[:, 1].reshape(1, n)
    cz = pos_c[:, 2].reshape(1, n)
    grid = (nq_pad // qblk,)
    qspec = pl.BlockSpec((qblk, 1), lambda i: (i, 0))
    cspec = pl.BlockSpec((1, n), lambda i: (0, 0))
    out = pl.pallas_call(
        functools.partial(_knn_body, k=k, qblk=qblk, exclude_self=exclude_self),
        grid=grid,
        in_specs=[qspec, qspec, qspec, cspec, cspec, cspec],
        out_specs=pl.BlockSpec((qblk, k), lambda i: (i, 0)),
        out_shape=jax.ShapeDtypeStruct((nq_pad, k), jnp.int32),
    )(qx, qy, qz, cx, cy, cz)
    return out[:nq]


def _knn_idx(pos_x, pos_y, k):
    return _knn_pallas(pos_y, pos_x, k, exclude_self=False)


def _knn_self(pos, k):
    return _knn_pallas(pos, pos, k, exclude_self=True)


_SC_NC = 2
_SC_NS = 16
_SC_NW = _SC_NC * _SC_NS


def _sc_gather_kernel(table_hbm, idx_hbm, out_hbm, idx_v, rows_v, gsem, ssem,
                      *, nchunk, rows, b_per_w):
    wid = lax.axis_index("s") * _SC_NC + lax.axis_index("c")
    pltpu.sync_copy(idx_hbm.at[wid], idx_v)
    nbuf = rows_v.shape[0]
    base = wid * b_per_w

    def fire_gather(j, buf):
        return pltpu.async_copy(table_hbm.at[idx_v.at[j]], rows_v.at[buf], gsem)

    def fire_scatter(j, buf):
        return pltpu.async_copy(rows_v.at[buf],
                                out_hbm.at[pl.ds(base + j * rows, rows)], ssem)

    gh = [None] * nbuf
    sh = [None] * nbuf
    lag = max(0, nbuf - 1)
    for j in range(nchunk + lag):
        if j < nchunk:
            b = j % nbuf
            if sh[b] is not None:
                sh[b].wait()
                sh[b] = None
            gh[b] = fire_gather(j, b)
        jc = j - lag
        if 0 <= jc:
            bc = jc % nbuf
            gh[bc].wait()
            sh[bc] = fire_scatter(jc, bc)
    for b in range(nbuf):
        if sh[b] is not None:
            sh[b].wait()


def _sc_gather(table, idx):
    v, d = table.shape
    d_pad = _round_up(d, 128)
    if d_pad != d:
        table = jnp.pad(table, ((0, 0), (0, d_pad - d)))
    rows = 8
    for r in (128, 64, 32, 16):
        if 4 * r * d_pad * 4 <= 420 * 1024:
            rows = r
            break
    bmult = _SC_NW * rows
    b = idx.shape[0]
    b_pad = _round_up(b, bmult)
    idx_p = jnp.pad(idx.astype(jnp.int32), (0, b_pad - b))
    b_per_w = b_pad // _SC_NW
    nchunk = b_per_w // rows
    idx3 = idx_p.reshape(_SC_NW, nchunk, rows)
    nbuf = min(4, nchunk)
    mesh = plsc.VectorSubcoreMesh(core_axis_name="c", subcore_axis_name="s")
    kfn = pl.kernel(
        functools.partial(_sc_gather_kernel, nchunk=nchunk, rows=rows,
                          b_per_w=b_per_w),
        mesh=mesh,
        out_type=jax.ShapeDtypeStruct((b_pad, d_pad), jnp.float32),
        scratch_types=[
            pltpu.VMEM((nchunk, rows), jnp.int32),
            pltpu.VMEM((nbuf, rows, d_pad), jnp.float32),
            pltpu.SemaphoreType.DMA,
            pltpu.SemaphoreType.DMA,
        ],
    )
    out = kfn(table, idx3)
    return out[:, :d] if d_pad != d else out


def _fps_pallas_body(px_ref, py_ref, pz_ref, idx_ref, dist_scratch, n_sample):
    px = px_ref[...]
    py = py_ref[...]
    pz = pz_ref[...]
    d0 = (px - px[0]) ** 2 + (py - py[0]) ** 2 + (pz - pz[0]) ** 2
    dist_scratch[...] = d0
    idx_ref[0, 0] = 0

    n = px.shape[0]
    iota = jax.lax.broadcasted_iota(jnp.int32, (n,), 0)

    def body(i, _):
        dists = dist_scratch[...]
        nxt = jnp.argmax(dists).astype(jnp.int32)
        idx_ref[0, i] = nxt
        sel = iota == nxt
        qx = jnp.sum(jnp.where(sel, px, 0.0))
        qy = jnp.sum(jnp.where(sel, py, 0.0))
        qz = jnp.sum(jnp.where(sel, pz, 0.0))
        d = (px - qx) ** 2 + (py - qy) ** 2 + (pz - qz) ** 2
        dist_scratch[...] = jnp.minimum(dists, d)
        return 0

    jax.lax.fori_loop(1, n_sample, body, 0)


def _fps(pos, n_sample):
    n = pos.shape[0]
    out = pl.pallas_call(
        functools.partial(_fps_pallas_body, n_sample=n_sample),
        out_shape=jax.ShapeDtypeStruct((1, n_sample), jnp.int32),
        out_specs=pl.BlockSpec(memory_space=pltpu.SMEM),
        scratch_shapes=[pltpu.VMEM((n,), jnp.float32)],
    )(pos[:, 0], pos[:, 1], pos[:, 2])
    return out[0]


def _edge_softmax_dense(alpha):
    amax = alpha.max(axis=1, keepdims=True)
    e = jnp.exp(alpha - amax)
    s = e.sum(axis=1, keepdims=True)
    return e / (s + 1e-16)


def _pt_conv(p, x, pos, nbr):
    n, d = x.shape
    a_src = x @ p['lin_src']
    a_dst = x @ p['lin_dst']
    xv = x @ p['lin']

    pos_pad = jnp.pad(pos, ((0, 0), (0, 13)))
    table = jnp.concatenate([a_src, xv, pos_pad], axis=1)
    g = _sc_gather(table, nbr.reshape(-1))[:n * K].reshape(n, K, 2 * d + 16)
    a_srcg = jnp.concatenate([g[:, :, :d], a_src[:, None, :]], axis=1)
    xvg = jnp.concatenate([g[:, :, d:2 * d], xv[:, None, :]], axis=1)
    pos_src = jnp.concatenate([g[:, :, 2 * d:2 * d + 3], pos[:, None, :]], axis=1)

    rel = pos[:, None, :] - pos_src
    delta = _mlp2_relu(p['pos_nn'], rel)
    alpha = a_dst[:, None, :] - a_srcg + delta
    alpha = _mlp2_relu(p['attn_nn'], alpha)
    alpha = _edge_softmax_dense(alpha)
    msg = alpha * (xvg + delta)
    return msg.sum(axis=1)


def _tblock(p, x, pos, nbr):
    x = jax.nn.relu(_lin(p['lin_in'], x))
    x = _pt_conv(p, x, pos, nbr)
    return jax.nn.relu(_lin(p['lin_out'], x))


def _tdown(p, x, pos, k):
    n_sample = int(math.ceil(pos.shape[0] * RATIO))
    idc = _fps(pos, n_sample)
    pos_pad = jnp.pad(pos, ((0, 0), (0, 13)))
    pos_s = _sc_gather(pos_pad, idc)[:n_sample, :3]
    nbr = _knn_idx(pos, pos_s, k)
    x = _lin_bn_relu(p, x)
    gathered = _sc_gather(x, nbr.reshape(-1))[:n_sample * k]
    x_out = gathered.reshape(n_sample, k, -1).max(axis=1)
    return x_out, pos_s


def kernel(x, pos, params):
    x = _lin_bn_relu(params['mlp_input'], x)
    nbr = _knn_self(pos, K)
    x = _tblock(params['tb0'], x, pos, nbr)
    for i in range(len(DIM_MODEL) - 1):
        x, pos = _tdown(params['td%d' % i], x, pos, K)
        nbr = _knn_self(pos, K)
        x = _tblock(params['tb%d' % (i + 1)], x, pos, nbr)
    x = x.mean(axis=0, keepdims=True)
    po = params['mlp_out']
    h = jax.nn.relu(x @ po['W1'] + po['b1'])
    out = h @ po['W2'] + po['b2']
    return jax.nn.log_softmax(out, axis=-1)


# speedup vs baseline: 1.0072x; 1.0072x over previous
import math
import functools

import jax
import jax.numpy as jnp
from jax import lax
from jax.experimental import pallas as pl
from jax.experimental.pallas import tpu as pltpu
from jax.experimental.pallas import tpu_sc as plsc

N_POINTS = 10000
IN_CH = 3
OUT_CH = 40
DIM_MODEL = [32, 64, 128, 256, 512]
K = 16
RATIO = 0.25


def _lin(p, x):
    return x @ p['W'] + p['b']


def _bn(p, x, eps=1e-5):
    mu = x.mean(0)
    var = x.var(0)
    return (x - mu) / jnp.sqrt(var + eps) * p['gamma'] + p['beta']


def _lin_bn_relu(p, x):
    return jax.nn.relu(_bn(p, x @ p['W'] + p['b']))


def _mlp2_relu(p, x):
    h = jax.nn.relu(x @ p['W1'] + p['b1'])
    return jax.nn.relu(h @ p['W2'] + p['b2'])


import numpy as np

_INF = np.float32(np.inf)
_BIG_I32 = np.int32(2**31 - 1)


def _knn_body(qx_ref, qy_ref, qz_ref, cx_ref, cy_ref, cz_ref, out_ref,
              *, k, qblk, exclude_self):
    n = cx_ref.shape[1]
    qx = qx_ref[...]
    qy = qy_ref[...]
    qz = qz_ref[...]
    cx = cx_ref[...]
    cy = cy_ref[...]
    cz = cz_ref[...]
    d = (qx - cx) ** 2 + (qy - cy) ** 2 + (qz - cz) ** 2
    ci = jax.lax.broadcasted_iota(jnp.int32, (qblk, n), 1)
    if exclude_self:
        s = pl.program_id(0) * qblk
        qi = jax.lax.broadcasted_iota(jnp.int32, (qblk, n), 0)
        d = jnp.where(ci == qi + s, _INF, d)
    for r in range(k):
        m = jnp.min(d, axis=1, keepdims=True)
        idx = jnp.min(jnp.where(d == m, ci, _BIG_I32), axis=1, keepdims=True)
        out_ref[:, r:r + 1] = idx
        d = jnp.where(ci == idx, _INF, d)


def _round_up(v, m):
    return ((v + m - 1) // m) * m


def _knn_pallas(pos_q, pos_c, k, exclude_self):
    nq = pos_q.shape[0]
    n = pos_c.shape[0]
    qblk = 512 if nq >= 512 else _round_up(nq, 8)
    nq_pad = _round_up(nq, qblk)
    q = jnp.pad(pos_q, ((0, nq_pad - nq), (0, 0)))
    qx = q[:, 0:1]
    qy = q[:, 1:2]
    qz = q[:, 2:3]
    cx = pos_c[:, 0].reshape(1, n)
    cy = pos_c[:, 1].reshape(1, n)
    cz = pos_c[:, 2].reshape(1, n)
    grid = (nq_pad // qblk,)
    qspec = pl.BlockSpec((qblk, 1), lambda i: (i, 0))
    cspec = pl.BlockSpec((1, n), lambda i: (0, 0))
    out = pl.pallas_call(
        functools.partial(_knn_body, k=k, qblk=qblk, exclude_self=exclude_self),
        grid=grid,
        in_specs=[qspec, qspec, qspec, cspec, cspec, cspec],
        out_specs=pl.BlockSpec((qblk, k), lambda i: (i, 0)),
        out_shape=jax.ShapeDtypeStruct((nq_pad, k), jnp.int32),
    )(qx, qy, qz, cx, cy, cz)
    return out[:nq]


def _knn_idx(pos_x, pos_y, k):
    return _knn_pallas(pos_y, pos_x, k, exclude_self=False)


def _knn_self(pos, k):
    return _knn_pallas(pos, pos, k, exclude_self=True)


_SC_NC = 2
_SC_NS = 16
_SC_NW = _SC_NC * _SC_NS


def _sc_gather_kernel(table_hbm, idx_hbm, out_hbm, idx_v, rows_v, gsem, ssem,
                      *, nchunk, rows, b_per_w):
    wid = lax.axis_index("s") * _SC_NC + lax.axis_index("c")
    pltpu.sync_copy(idx_hbm.at[wid], idx_v)
    nbuf = rows_v.shape[0]
    base = wid * b_per_w

    def fire_gather(j, buf):
        return pltpu.async_copy(table_hbm.at[idx_v.at[j]], rows_v.at[buf], gsem)

    def fire_scatter(j, buf):
        return pltpu.async_copy(rows_v.at[buf],
                                out_hbm.at[pl.ds(base + j * rows, rows)], ssem)

    gh = [None] * nbuf
    sh = [None] * nbuf
    lag = max(0, nbuf - 1)
    for j in range(nchunk + lag):
        if j < nchunk:
            b = j % nbuf
            if sh[b] is not None:
                sh[b].wait()
                sh[b] = None
            gh[b] = fire_gather(j, b)
        jc = j - lag
        if 0 <= jc:
            bc = jc % nbuf
            gh[bc].wait()
            sh[bc] = fire_scatter(jc, bc)
    for b in range(nbuf):
        if sh[b] is not None:
            sh[b].wait()


def _sc_gather(table, idx):
    v, d = table.shape
    d_pad = _round_up(d, 128)
    if d_pad != d:
        table = jnp.pad(table, ((0, 0), (0, d_pad - d)))
    rows = 8
    for r in (128, 64, 32, 16):
        if 4 * r * d_pad * 4 <= 420 * 1024:
            rows = r
            break
    bmult = _SC_NW * rows
    b = idx.shape[0]
    b_pad = _round_up(b, bmult)
    idx_p = jnp.pad(idx.astype(jnp.int32), (0, b_pad - b))
    b_per_w = b_pad // _SC_NW
    nchunk = b_per_w // rows
    idx3 = idx_p.reshape(_SC_NW, nchunk, rows)
    nbuf = min(4, nchunk)
    mesh = plsc.VectorSubcoreMesh(core_axis_name="c", subcore_axis_name="s")
    kfn = pl.kernel(
        functools.partial(_sc_gather_kernel, nchunk=nchunk, rows=rows,
                          b_per_w=b_per_w),
        mesh=mesh,
        out_type=jax.ShapeDtypeStruct((b_pad, d_pad), jnp.float32),
        scratch_types=[
            pltpu.VMEM((nchunk, rows), jnp.int32),
            pltpu.VMEM((nbuf, rows, d_pad), jnp.float32),
            pltpu.SemaphoreType.DMA,
            pltpu.SemaphoreType.DMA,
        ],
    )
    out = kfn(table, idx3)
    return out[:, :d] if d_pad != d else out


def _fps_pallas_body(px_ref, py_ref, pz_ref, idx_ref, dist_scratch, n_sample):
    px = px_ref[...]
    py = py_ref[...]
    pz = pz_ref[...]
    d0 = (px - px[0]) ** 2 + (py - py[0]) ** 2 + (pz - pz[0]) ** 2
    dist_scratch[...] = d0
    idx_ref[0, 0] = 0

    n = px.shape[0]
    iota = jax.lax.broadcasted_iota(jnp.int32, (n,), 0)

    def body(i, _):
        dists = dist_scratch[...]
        nxt = jnp.argmax(dists).astype(jnp.int32)
        idx_ref[0, i] = nxt
        sel = iota == nxt
        qx = jnp.sum(jnp.where(sel, px, 0.0))
        qy = jnp.sum(jnp.where(sel, py, 0.0))
        qz = jnp.sum(jnp.where(sel, pz, 0.0))
        d = (px - qx) ** 2 + (py - qy) ** 2 + (pz - qz) ** 2
        dist_scratch[...] = jnp.minimum(dists, d)
        return 0

    jax.lax.fori_loop(1, n_sample, body, 0)


def _fps(pos, n_sample):
    n = pos.shape[0]
    out = pl.pallas_call(
        functools.partial(_fps_pallas_body, n_sample=n_sample),
        out_shape=jax.ShapeDtypeStruct((1, n_sample), jnp.int32),
        out_specs=pl.BlockSpec(memory_space=pltpu.SMEM),
        scratch_shapes=[pltpu.VMEM((n,), jnp.float32)],
    )(pos[:, 0], pos[:, 1], pos[:, 2])
    return out[0]


def _edge_softmax_dense(alpha):
    amax = alpha.max(axis=1, keepdims=True)
    e = jnp.exp(alpha - amax)
    s = e.sum(axis=1, keepdims=True)
    return e / (s + 1e-16)


def _pt_conv(p, x, pos, nbr):
    n, d = x.shape
    a_src = x @ p['lin_src']
    a_dst = x @ p['lin_dst']
    xv = x @ p['lin']

    pos_pad = jnp.pad(pos, ((0, 0), (0, 13)))
    table = jnp.concatenate([a_src, xv, pos_pad], axis=1)
    g = _sc_gather(table, nbr.reshape(-1))[:n * K].reshape(n, K, 2 * d + 16)
    a_srcg = jnp.concatenate([g[:, :, :d], a_src[:, None, :]], axis=1)
    xvg = jnp.concatenate([g[:, :, d:2 * d], xv[:, None, :]], axis=1)
    pos_src = jnp.concatenate([g[:, :, 2 * d:2 * d + 3], pos[:, None, :]], axis=1)

    rel = pos[:, None, :] - pos_src
    delta = _mlp2_relu(p['pos_nn'], rel)
    alpha = a_dst[:, None, :] - a_srcg + delta
    alpha = _mlp2_relu(p['attn_nn'], alpha)
    alpha = _edge_softmax_dense(alpha)
    msg = alpha * (xvg + delta)
    return msg.sum(axis=1)


def _tblock(p, x, pos, nbr):
    x = jax.nn.relu(_lin(p['lin_in'], x))
    x = _pt_conv(p, x, pos, nbr)
    return jax.nn.relu(_lin(p['lin_out'], x))


def _tdown(p, x, pos, k):
    n_sample = int(math.ceil(pos.shape[0] * RATIO))
    idc = _fps(pos, n_sample)
    pos_pad = jnp.pad(pos, ((0, 0), (0, 13)))
    pos_s = _sc_gather(pos_pad, idc)[:n_sample, :3]
    nbr = _knn_idx(pos, pos_s, k)
    x = _lin_bn_relu(p, x)
    gathered = _sc_gather(x, nbr.reshape(-1))[:n_sample * k]
    x_out = gathered.reshape(n_sample, k, -1).max(axis=1)
    return x_out, pos_s


def kernel(x, pos, params):
    x = _lin_bn_relu(params['mlp_input'], x)
    nbr = _knn_self(pos, K)
    x = _tblock(params['tb0'], x, pos, nbr)
    for i in range(len(DIM_MODEL) - 1):
        x, pos = _tdown(params['td%d' % i], x, pos, K)
        nbr = _knn_self(pos, K)
        x = _tblock(params['tb%d' % (i + 1)], x, pos, nbr)
    x = x.mean(axis=0, keepdims=True)
    po = params['mlp_out']
    h = jax.nn.relu(x @ po['W1'] + po['b1'])
    out = h @ po['W2'] + po['b2']
    return jax.nn.log_softmax(out, axis=-1)
